# Initial kernel scaffold; baseline (speedup 1.0000x reference)
import jax
import jax.numpy as jnp
from jax.experimental import pallas as pl


def kernel(x, edge_index, batch, W1, b1, W2, b2, Wl, bl):
    raise NotImplementedError("write your pallas kernel here")


# speedup vs baseline: 16.4489x; 16.4489x over previous
import functools

import jax
import jax.numpy as jnp
from jax import lax
from jax.experimental import pallas as pl
from jax.experimental.pallas import tpu as pltpu
from jax.experimental.pallas import tpu_sc as plsc

N = 10000
E = 320000
D = 128
NG = 64

NC = 2
NS = 16
NW = NC * NS
CHUNK = 128
NCH = E // CHUNK

NPAD = 10240
TROWS = NPAD // NS
ZR = 128

BLK = 1000
NB = N // BLK

DSL = (TROWS // 8) * 8
DSL_LAST = N - DSL * (NS - 1)


def _wid():
    c = lax.axis_index("c")
    s = lax.axis_index("s")
    return s * NC + c, c, s


def _deg_body(dst_hbm, out_hbm, ones_v, didx, zbuf, acc1):
    wid, c, s = _wid()
    for j in range(8):
        ones_v[pl.ds(j * 16, 16)] = jnp.ones((16,), jnp.float32)
    for j in range(DSL_LAST // 16):
        zbuf[pl.ds(j * 16, 16)] = jnp.zeros((16,), jnp.float32)

    @pl.when(s < NS - 1)
    def _():
        pltpu.sync_copy(zbuf.at[pl.ds(0, DSL)], acc1.at[pl.ds(s * DSL, DSL)])

    @pl.when(s == NS - 1)
    def _():
        pltpu.sync_copy(zbuf, acc1.at[pl.ds((NS - 1) * DSL, DSL_LAST)])

    plsc.subcore_barrier()

    nk = NCH // NW + jnp.where(wid < NCH % NW, 1, 0)

    def body(k, carry):
        off = (wid + k * NW) * CHUNK
        pltpu.sync_copy(dst_hbm.at[pl.ds(off, CHUNK)], didx)
        pltpu.sync_copy(ones_v, acc1.at[didx], add=True)
        return carry

    lax.fori_loop(0, nk, body, 0)
    plsc.subcore_barrier()

    @pl.when(s < NS - 1)
    def _():
        pltpu.sync_copy(acc1.at[pl.ds(s * DSL, DSL)], zbuf.at[pl.ds(0, DSL)])
        pltpu.sync_copy(zbuf.at[pl.ds(0, DSL)],
                        out_hbm.at[pl.ds(c * N + s * DSL, DSL)])

    @pl.when(s == NS - 1)
    def _():
        pltpu.sync_copy(acc1.at[pl.ds((NS - 1) * DSL, DSL_LAST)], zbuf)
        pltpu.sync_copy(zbuf,
                        out_hbm.at[pl.ds(c * N + (NS - 1) * DSL, DSL_LAST)])


_deg_call = pl.kernel(
    _deg_body,
    out_type=jax.ShapeDtypeStruct((NC * N,), jnp.float32),
    mesh=plsc.VectorSubcoreMesh(core_axis_name="c", subcore_axis_name="s"),
    scratch_types=[
        pltpu.VMEM((CHUNK,), jnp.float32),
        pltpu.VMEM((CHUNK,), jnp.int32),
        pltpu.VMEM((DSL_LAST,), jnp.float32),
        pltpu.VMEM_SHARED((N,), jnp.float32),
    ],
)


def _scat_body(src_hbm, dst_hbm, table_hbm, out_hbm,
               sidx, didx, rows, zrows, acc, gsem):
    wid, c, s = _wid()

    def zr(i, carry):
        for j in range(D // 16):
            zrows[i, pl.ds(j * 16, 16)] = jnp.zeros((16,), jnp.float32)
        return carry

    lax.fori_loop(0, ZR, zr, 0)
    for j in range(TROWS // ZR):
        pltpu.sync_copy(zrows, acc.at[pl.ds(s * TROWS + j * ZR, ZR)])
    plsc.subcore_barrier()

    nk = NCH // NW + jnp.where(wid < NCH % NW, 1, 0)

    def body(k, carry):
        off = (wid + k * NW) * CHUNK
        pltpu.sync_copy(src_hbm.at[pl.ds(off, CHUNK)], sidx)
        pltpu.sync_copy(dst_hbm.at[pl.ds(off, CHUNK)], didx)
        pltpu.async_copy(table_hbm.at[sidx], rows, gsem).wait()
        pltpu.sync_copy(rows, acc.at[didx], add=True)
        return carry

    lax.fori_loop(0, nk, body, 0)
    plsc.subcore_barrier()
    for j in range(TROWS // ZR):
        pltpu.sync_copy(acc.at[pl.ds(s * TROWS + j * ZR, ZR)], zrows)
        pltpu.sync_copy(zrows,
                        out_hbm.at[pl.ds(c * NPAD + s * TROWS + j * ZR, ZR)])


_scat_call = pl.kernel(
    _scat_body,
    out_type=jax.ShapeDtypeStruct((NC * NPAD, D), jnp.float32),
    mesh=plsc.VectorSubcoreMesh(core_axis_name="c", subcore_axis_name="s"),
    scratch_types=[
        pltpu.VMEM((CHUNK,), jnp.int32),
        pltpu.VMEM((CHUNK,), jnp.int32),
        pltpu.VMEM((CHUNK, D), jnp.float32),
        pltpu.VMEM((ZR, D), jnp.float32),
        pltpu.VMEM_SHARED((NPAD, D), jnp.float32),
        pltpu.SemaphoreType.DMA,
    ],
)


def _dinv(degp_ref):
    return lax.rsqrt(jnp.sum(degp_ref[...], axis=1, keepdims=True) + 1.0)


def _tc1_body(x_ref, w_ref, degp_ref, g1_ref):
    dinv = _dinv(degp_ref)
    h = jnp.dot(x_ref[...], w_ref[...], preferred_element_type=jnp.float32,
                precision=lax.Precision.HIGHEST)
    g1_ref[...] = h * dinv


def _tc2_body(sp_ref, g1_ref, degp_ref, b1_ref, w2_ref, g2_ref):
    dinv = _dinv(degp_ref)
    ssum = sp_ref[0] + sp_ref[1] + g1_ref[...]
    out1 = jnp.maximum(ssum * dinv + b1_ref[...], 0.0)
    h2 = jnp.dot(out1, w2_ref[...], preferred_element_type=jnp.float32,
                 precision=lax.Precision.HIGHEST)
    g2_ref[...] = h2 * dinv


def _tc3_body(sp_ref, g2_ref, degp_ref, b2_ref, batch_ref, wl_ref, bl_ref,
              out_ref, sums, counts):
    i = pl.program_id(0)
    dinv = _dinv(degp_ref)
    ssum = sp_ref[0] + sp_ref[1] + g2_ref[...]
    h2 = jnp.maximum(ssum * dinv + b2_ref[...], 0.0)
    b = batch_ref[0, 0, :]
    gids = lax.broadcasted_iota(jnp.int32, (NG, BLK), 0)
    oh = (b[None, :] == gids).astype(jnp.float32)

    @pl.when(i == 0)
    def _():
        sums[...] = jnp.zeros_like(sums)
        counts[...] = jnp.zeros_like(counts)

    sums[...] += jnp.dot(oh, h2, preferred_element_type=jnp.float32,
                         precision=lax.Precision.HIGHEST)
    counts[...] += jnp.broadcast_to(jnp.sum(oh, axis=1)[:, None], (NG, D))

    @pl.when(i == pl.num_programs(0) - 1)
    def _():
        pooled = sums[...] / jnp.maximum(counts[...], 1.0)
        out_ref[...] = jnp.dot(pooled, wl_ref[...],
                               preferred_element_type=jnp.float32,
                               precision=lax.Precision.HIGHEST) + bl_ref[...]


_tc1_call = pl.pallas_call(
    _tc1_body,
    grid=(NB,),
    in_specs=[
        pl.BlockSpec((BLK, D), lambda i: (i, 0)),
        pl.BlockSpec((D, D), lambda i: (0, 0)),
        pl.BlockSpec((BLK, NC), lambda i: (i, 0)),
    ],
    out_specs=pl.BlockSpec((BLK, D), lambda i: (i, 0)),
    out_shape=jax.ShapeDtypeStruct((N, D), jnp.float32),
)

_tc2_call = pl.pallas_call(
    _tc2_body,
    grid=(NB,),
    in_specs=[
        pl.BlockSpec((NC, BLK, D), lambda i: (0, i, 0)),
        pl.BlockSpec((BLK, D), lambda i: (i, 0)),
        pl.BlockSpec((BLK, NC), lambda i: (i, 0)),
        pl.BlockSpec((1, D), lambda i: (0, 0)),
        pl.BlockSpec((D, D), lambda i: (0, 0)),
    ],
    out_specs=pl.BlockSpec((BLK, D), lambda i: (i, 0)),
    out_shape=jax.ShapeDtypeStruct((N, D), jnp.float32),
)

_tc3_call = pl.pallas_call(
    _tc3_body,
    grid=(NB,),
    in_specs=[
        pl.BlockSpec((NC, BLK, D), lambda i: (0, i, 0)),
        pl.BlockSpec((BLK, D), lambda i: (i, 0)),
        pl.BlockSpec((BLK, NC), lambda i: (i, 0)),
        pl.BlockSpec((1, D), lambda i: (0, 0)),
        pl.BlockSpec((1, 1, BLK), lambda i: (i, 0, 0)),
        pl.BlockSpec((D, D), lambda i: (0, 0)),
        pl.BlockSpec((1, D), lambda i: (0, 0)),
    ],
    out_specs=pl.BlockSpec((NG, D), lambda i: (0, 0)),
    out_shape=jax.ShapeDtypeStruct((NG, D), jnp.float32),
    scratch_shapes=[
        pltpu.VMEM((NG, D), jnp.float32),
        pltpu.VMEM((NG, D), jnp.float32),
    ],
)


def kernel(x, edge_index, batch, W1, b1, W2, b2, Wl, bl):
    src = edge_index[0]
    dst = edge_index[1]
    degp = _deg_call(dst).reshape(NC, N).T
    g1 = _tc1_call(x, W1, degp)
    s1 = _scat_call(src, dst, g1).reshape(NC, NPAD, D)
    g2 = _tc2_call(s1, g1, degp, b1.reshape(1, D), W2)
    s2 = _scat_call(src, dst, g2).reshape(NC, NPAD, D)
    out = _tc3_call(s2, g2, degp, b2.reshape(1, D), batch.reshape(NB, 1, BLK),
                    Wl, bl.reshape(1, D))
    return out


# speedup vs baseline: 18.6401x; 1.1332x over previous
import functools

import jax
import jax.numpy as jnp
from jax import lax
from jax.experimental import pallas as pl
from jax.experimental.pallas import tpu as pltpu
from jax.experimental.pallas import tpu_sc as plsc

N = 10000
E = 320000
D = 128
NG = 64

NC = 2
NS = 16
NW = NC * NS
CHUNK = 128
NCH = E // CHUNK

NPAD = 10240
TROWS = NPAD // NS
ZR = 128

BLK = 1000
NB = N // BLK

DSL = (TROWS // 8) * 8
DEG_DSL = ((N // NS) // 8) * 8
DEG_DSL_LAST = N - DEG_DSL * (NS - 1)


def _wid():
    c = lax.axis_index("c")
    s = lax.axis_index("s")
    return s * NC + c, c, s


def _deg_body(dst_hbm, out_hbm, ones_v, didx, zbuf, acc1):
    wid, c, s = _wid()
    for j in range(8):
        ones_v[pl.ds(j * 16, 16)] = jnp.ones((16,), jnp.float32)
    for j in range(DEG_DSL_LAST // 16):
        zbuf[pl.ds(j * 16, 16)] = jnp.zeros((16,), jnp.float32)

    @pl.when(s < NS - 1)
    def _():
        pltpu.sync_copy(zbuf.at[pl.ds(0, DEG_DSL)],
                        acc1.at[pl.ds(s * DEG_DSL, DEG_DSL)])

    @pl.when(s == NS - 1)
    def _():
        pltpu.sync_copy(zbuf, acc1.at[pl.ds((NS - 1) * DEG_DSL,
                                            DEG_DSL_LAST)])

    plsc.subcore_barrier()

    nk = NCH // NW + jnp.where(wid < NCH % NW, 1, 0)

    def body(k, carry):
        off = (wid + k * NW) * CHUNK
        pltpu.sync_copy(dst_hbm.at[pl.ds(off, CHUNK)], didx)
        pltpu.sync_copy(ones_v, acc1.at[didx], add=True)
        return carry

    lax.fori_loop(0, nk, body, 0)
    plsc.subcore_barrier()

    @pl.when(s < NS - 1)
    def _():
        pltpu.sync_copy(acc1.at[pl.ds(s * DEG_DSL, DEG_DSL)],
                        zbuf.at[pl.ds(0, DEG_DSL)])
        pltpu.sync_copy(zbuf.at[pl.ds(0, DEG_DSL)],
                        out_hbm.at[pl.ds(c * N + s * DEG_DSL, DEG_DSL)])

    @pl.when(s == NS - 1)
    def _():
        pltpu.sync_copy(acc1.at[pl.ds((NS - 1) * DEG_DSL, DEG_DSL_LAST)],
                        zbuf)
        pltpu.sync_copy(zbuf,
                        out_hbm.at[pl.ds(c * N + (NS - 1) * DEG_DSL,
                                         DEG_DSL_LAST)])


_deg_call = pl.kernel(
    _deg_body,
    out_type=jax.ShapeDtypeStruct((NC * N,), jnp.float32),
    mesh=plsc.VectorSubcoreMesh(core_axis_name="c", subcore_axis_name="s"),
    scratch_types=[
        pltpu.VMEM((CHUNK,), jnp.float32),
        pltpu.VMEM((CHUNK,), jnp.int32),
        pltpu.VMEM((DEG_DSL_LAST,), jnp.float32),
        pltpu.VMEM_SHARED((N,), jnp.float32),
    ],
)


def _scat_body(src_hbm, dst_hbm, table_hbm, out_hbm,
               sidx, didx, rows, zrows, acc, gsem, sem_s, sem_d):
    wid, c, s = _wid()

    def zr(i, carry):
        for j in range(D // 16):
            zrows[i, pl.ds(j * 16, 16)] = jnp.zeros((16,), jnp.float32)
        return carry

    lax.fori_loop(0, ZR, zr, 0)
    for j in range(TROWS // ZR):
        pltpu.sync_copy(zrows, acc.at[pl.ds(s * TROWS + j * ZR, ZR)])
    plsc.subcore_barrier()

    nk = NCH // NW + jnp.where(wid < NCH % NW, 1, 0)

    def body(k, carry):
        off = (wid + k * NW) * CHUNK
        cp_s = pltpu.async_copy(src_hbm.at[pl.ds(off, CHUNK)], sidx, sem_s)
        cp_d = pltpu.async_copy(dst_hbm.at[pl.ds(off, CHUNK)], didx, sem_d)
        cp_s.wait()
        gather = pltpu.async_copy(table_hbm.at[sidx], rows, gsem)
        cp_d.wait()
        gather.wait()
        pltpu.sync_copy(rows, acc.at[didx], add=True)
        return carry

    lax.fori_loop(0, nk, body, 0)
    plsc.subcore_barrier()
    for j in range(TROWS // ZR):
        pltpu.sync_copy(acc.at[pl.ds(s * TROWS + j * ZR, ZR)], zrows)
        pltpu.sync_copy(zrows,
                        out_hbm.at[pl.ds(c * NPAD + s * TROWS + j * ZR, ZR)])


_scat_call = pl.kernel(
    _scat_body,
    out_type=jax.ShapeDtypeStruct((NC * NPAD, D), jnp.float32),
    mesh=plsc.VectorSubcoreMesh(core_axis_name="c", subcore_axis_name="s"),
    scratch_types=[
        pltpu.VMEM((CHUNK,), jnp.int32),
        pltpu.VMEM((CHUNK,), jnp.int32),
        pltpu.VMEM((CHUNK, D), jnp.float32),
        pltpu.VMEM((ZR, D), jnp.float32),
        pltpu.VMEM_SHARED((NPAD, D), jnp.float32),
        pltpu.SemaphoreType.DMA,
        pltpu.SemaphoreType.DMA,
        pltpu.SemaphoreType.DMA,
    ],
)


def _dinv(degp_ref):
    return lax.rsqrt(jnp.sum(degp_ref[...], axis=1, keepdims=True) + 1.0)


def _tc1_body(x_ref, w_ref, degp_ref, g1_ref):
    dinv = _dinv(degp_ref)
    h = jnp.dot(x_ref[...], w_ref[...], preferred_element_type=jnp.float32,
                precision=lax.Precision.HIGHEST)
    g1_ref[...] = h * dinv


def _tc2_body(sp_ref, g1_ref, degp_ref, b1_ref, w2_ref, g2_ref):
    dinv = _dinv(degp_ref)
    ssum = sp_ref[0] + sp_ref[1] + g1_ref[...]
    out1 = jnp.maximum(ssum * dinv + b1_ref[...], 0.0)
    h2 = jnp.dot(out1, w2_ref[...], preferred_element_type=jnp.float32,
                 precision=lax.Precision.HIGHEST)
    g2_ref[...] = h2 * dinv


def _tc3_body(sp_ref, g2_ref, degp_ref, b2_ref, batch_ref, wl_ref, bl_ref,
              out_ref, sums, counts):
    i = pl.program_id(0)
    dinv = _dinv(degp_ref)
    ssum = sp_ref[0] + sp_ref[1] + g2_ref[...]
    h2 = jnp.maximum(ssum * dinv + b2_ref[...], 0.0)
    b = batch_ref[0, 0, :]
    gids = lax.broadcasted_iota(jnp.int32, (NG, BLK), 0)
    oh = (b[None, :] == gids).astype(jnp.float32)

    @pl.when(i == 0)
    def _():
        sums[...] = jnp.zeros_like(sums)
        counts[...] = jnp.zeros_like(counts)

    sums[...] += jnp.dot(oh, h2, preferred_element_type=jnp.float32,
                         precision=lax.Precision.HIGHEST)
    counts[...] += jnp.broadcast_to(jnp.sum(oh, axis=1)[:, None], (NG, D))

    @pl.when(i == pl.num_programs(0) - 1)
    def _():
        pooled = sums[...] / jnp.maximum(counts[...], 1.0)
        out_ref[...] = jnp.dot(pooled, wl_ref[...],
                               preferred_element_type=jnp.float32,
                               precision=lax.Precision.HIGHEST) + bl_ref[...]


_tc1_call = pl.pallas_call(
    _tc1_body,
    grid=(NB,),
    in_specs=[
        pl.BlockSpec((BLK, D), lambda i: (i, 0)),
        pl.BlockSpec((D, D), lambda i: (0, 0)),
        pl.BlockSpec((BLK, NC), lambda i: (i, 0)),
    ],
    out_specs=pl.BlockSpec((BLK, D), lambda i: (i, 0)),
    out_shape=jax.ShapeDtypeStruct((N, D), jnp.float32),
)

_tc2_call = pl.pallas_call(
    _tc2_body,
    grid=(NB,),
    in_specs=[
        pl.BlockSpec((NC, BLK, D), lambda i: (0, i, 0)),
        pl.BlockSpec((BLK, D), lambda i: (i, 0)),
        pl.BlockSpec((BLK, NC), lambda i: (i, 0)),
        pl.BlockSpec((1, D), lambda i: (0, 0)),
        pl.BlockSpec((D, D), lambda i: (0, 0)),
    ],
    out_specs=pl.BlockSpec((BLK, D), lambda i: (i, 0)),
    out_shape=jax.ShapeDtypeStruct((N, D), jnp.float32),
)

_tc3_call = pl.pallas_call(
    _tc3_body,
    grid=(NB,),
    in_specs=[
        pl.BlockSpec((NC, BLK, D), lambda i: (0, i, 0)),
        pl.BlockSpec((BLK, D), lambda i: (i, 0)),
        pl.BlockSpec((BLK, NC), lambda i: (i, 0)),
        pl.BlockSpec((1, D), lambda i: (0, 0)),
        pl.BlockSpec((1, 1, BLK), lambda i: (i, 0, 0)),
        pl.BlockSpec((D, D), lambda i: (0, 0)),
        pl.BlockSpec((1, D), lambda i: (0, 0)),
    ],
    out_specs=pl.BlockSpec((NG, D), lambda i: (0, 0)),
    out_shape=jax.ShapeDtypeStruct((NG, D), jnp.float32),
    scratch_shapes=[
        pltpu.VMEM((NG, D), jnp.float32),
        pltpu.VMEM((NG, D), jnp.float32),
    ],
)


def kernel(x, edge_index, batch, W1, b1, W2, b2, Wl, bl):
    src = edge_index[0]
    dst = edge_index[1]
    degp = _deg_call(dst).reshape(NC, N).T
    g1 = _tc1_call(x, W1, degp)
    s1 = _scat_call(src, dst, g1).reshape(NC, NPAD, D)
    g2 = _tc2_call(s1, g1, degp, b1.reshape(1, D), W2)
    s2 = _scat_call(src, dst, g2).reshape(NC, NPAD, D)
    out = _tc3_call(s2, g2, degp, b2.reshape(1, D), batch.reshape(NB, 1, BLK),
                    Wl, bl.reshape(1, D))
    return out


# speedup vs baseline: 22.0953x; 1.1854x over previous
import functools

import jax
import jax.numpy as jnp
from jax import lax
from jax.experimental import pallas as pl
from jax.experimental.pallas import tpu as pltpu
from jax.experimental.pallas import tpu_sc as plsc

N = 10000
E = 320000
D = 128
NG = 64

NC = 2
NS = 16
NW = NC * NS
CHUNK = 128
NCH = E // CHUNK

NPAD = 10240
TROWS = NPAD // NS
ZR = 128

BLK = 1000
NB = N // BLK

DSL = (TROWS // 8) * 8
DEG_DSL = ((N // NS) // 8) * 8
DEG_DSL_LAST = N - DEG_DSL * (NS - 1)


def _wid():
    c = lax.axis_index("c")
    s = lax.axis_index("s")
    return s * NC + c, c, s


def _deg_body(dst_hbm, out_hbm, ones_v, didx, zbuf, acc1):
    wid, c, s = _wid()
    for j in range(8):
        ones_v[pl.ds(j * 16, 16)] = jnp.ones((16,), jnp.float32)
    for j in range(DEG_DSL_LAST // 16):
        zbuf[pl.ds(j * 16, 16)] = jnp.zeros((16,), jnp.float32)

    @pl.when(s < NS - 1)
    def _():
        pltpu.sync_copy(zbuf.at[pl.ds(0, DEG_DSL)],
                        acc1.at[pl.ds(s * DEG_DSL, DEG_DSL)])

    @pl.when(s == NS - 1)
    def _():
        pltpu.sync_copy(zbuf, acc1.at[pl.ds((NS - 1) * DEG_DSL,
                                            DEG_DSL_LAST)])

    plsc.subcore_barrier()

    nk = NCH // NW + jnp.where(wid < NCH % NW, 1, 0)

    def body(k, carry):
        off = (wid + k * NW) * CHUNK
        pltpu.sync_copy(dst_hbm.at[pl.ds(off, CHUNK)], didx)
        pltpu.sync_copy(ones_v, acc1.at[didx], add=True)
        return carry

    lax.fori_loop(0, nk, body, 0)
    plsc.subcore_barrier()

    @pl.when(s < NS - 1)
    def _():
        pltpu.sync_copy(acc1.at[pl.ds(s * DEG_DSL, DEG_DSL)],
                        zbuf.at[pl.ds(0, DEG_DSL)])
        pltpu.sync_copy(zbuf.at[pl.ds(0, DEG_DSL)],
                        out_hbm.at[pl.ds(c * N + s * DEG_DSL, DEG_DSL)])

    @pl.when(s == NS - 1)
    def _():
        pltpu.sync_copy(acc1.at[pl.ds((NS - 1) * DEG_DSL, DEG_DSL_LAST)],
                        zbuf)
        pltpu.sync_copy(zbuf,
                        out_hbm.at[pl.ds(c * N + (NS - 1) * DEG_DSL,
                                         DEG_DSL_LAST)])


_deg_call = pl.kernel(
    _deg_body,
    out_type=jax.ShapeDtypeStruct((NC * N,), jnp.float32),
    mesh=plsc.VectorSubcoreMesh(core_axis_name="c", subcore_axis_name="s"),
    scratch_types=[
        pltpu.VMEM((CHUNK,), jnp.float32),
        pltpu.VMEM((CHUNK,), jnp.int32),
        pltpu.VMEM((DEG_DSL_LAST,), jnp.float32),
        pltpu.VMEM_SHARED((N,), jnp.float32),
    ],
)


def _scat_body(src_hbm, dst_hbm, table_hbm, out_hbm,
               sidx, didx, sidx2, didx2, rows, rows2, acc,
               gsem, sem_s, sem_d, gsem2, sem_s2, sem_d2):
    wid, c, s = _wid()

    def zr(i, carry):
        for j in range(D // 16):
            rows[i, pl.ds(j * 16, 16)] = jnp.zeros((16,), jnp.float32)
        return carry

    lax.fori_loop(0, ZR, zr, 0)
    for j in range(TROWS // ZR):
        pltpu.sync_copy(rows, acc.at[pl.ds(s * TROWS + j * ZR, ZR)])
    plsc.subcore_barrier()

    def body(i, carry):
        offa = (wid + (2 * i) * NW) * CHUNK
        offb = (wid + (2 * i + 1) * NW) * CHUNK
        cpa_s = pltpu.async_copy(src_hbm.at[pl.ds(offa, CHUNK)], sidx, sem_s)
        cpa_d = pltpu.async_copy(dst_hbm.at[pl.ds(offa, CHUNK)], didx, sem_d)
        cpb_s = pltpu.async_copy(src_hbm.at[pl.ds(offb, CHUNK)], sidx2,
                                 sem_s2)
        cpb_d = pltpu.async_copy(dst_hbm.at[pl.ds(offb, CHUNK)], didx2,
                                 sem_d2)
        cpa_s.wait()
        ga = pltpu.async_copy(table_hbm.at[sidx], rows, gsem)
        cpb_s.wait()
        gb = pltpu.async_copy(table_hbm.at[sidx2], rows2, gsem2)
        ga.wait()
        cpa_d.wait()
        pltpu.sync_copy(rows, acc.at[didx], add=True)
        gb.wait()
        cpb_d.wait()
        pltpu.sync_copy(rows2, acc.at[didx2], add=True)
        return carry

    lax.fori_loop(0, NCH // NW // 2, body, 0)

    @pl.when(wid < NCH % NW)
    def _():
        off = (wid + (NCH // NW) * NW) * CHUNK
        pltpu.sync_copy(src_hbm.at[pl.ds(off, CHUNK)], sidx)
        pltpu.sync_copy(dst_hbm.at[pl.ds(off, CHUNK)], didx)
        pltpu.async_copy(table_hbm.at[sidx], rows, gsem).wait()
        pltpu.sync_copy(rows, acc.at[didx], add=True)

    plsc.subcore_barrier()
    for j in range(TROWS // ZR):
        buf = rows if j % 2 == 0 else rows2
        pltpu.sync_copy(acc.at[pl.ds(s * TROWS + j * ZR, ZR)], buf)
        pltpu.sync_copy(buf,
                        out_hbm.at[pl.ds(c * NPAD + s * TROWS + j * ZR, ZR)])


_scat_call = pl.kernel(
    _scat_body,
    out_type=jax.ShapeDtypeStruct((NC * NPAD, D), jnp.float32),
    mesh=plsc.VectorSubcoreMesh(core_axis_name="c", subcore_axis_name="s"),
    scratch_types=[
        pltpu.VMEM((CHUNK,), jnp.int32),
        pltpu.VMEM((CHUNK,), jnp.int32),
        pltpu.VMEM((CHUNK,), jnp.int32),
        pltpu.VMEM((CHUNK,), jnp.int32),
        pltpu.VMEM((CHUNK, D), jnp.float32),
        pltpu.VMEM((CHUNK, D), jnp.float32),
        pltpu.VMEM_SHARED((NPAD, D), jnp.float32),
        pltpu.SemaphoreType.DMA,
        pltpu.SemaphoreType.DMA,
        pltpu.SemaphoreType.DMA,
        pltpu.SemaphoreType.DMA,
        pltpu.SemaphoreType.DMA,
        pltpu.SemaphoreType.DMA,
    ],
)


def _dinv(degp_ref):
    return lax.rsqrt(jnp.sum(degp_ref[...], axis=1, keepdims=True) + 1.0)


def _tc1_body(x_ref, w_ref, degp_ref, g1_ref):
    dinv = _dinv(degp_ref)
    h = jnp.dot(x_ref[...], w_ref[...], preferred_element_type=jnp.float32,
                precision=lax.Precision.HIGHEST)
    g1_ref[...] = h * dinv


def _tc2_body(sp_ref, g1_ref, degp_ref, b1_ref, w2_ref, g2_ref):
    dinv = _dinv(degp_ref)
    ssum = sp_ref[0] + sp_ref[1] + g1_ref[...]
    out1 = jnp.maximum(ssum * dinv + b1_ref[...], 0.0)
    h2 = jnp.dot(out1, w2_ref[...], preferred_element_type=jnp.float32,
                 precision=lax.Precision.HIGHEST)
    g2_ref[...] = h2 * dinv


def _tc3_body(sp_ref, g2_ref, degp_ref, b2_ref, batch_ref, wl_ref, bl_ref,
              out_ref, sums, counts):
    i = pl.program_id(0)
    dinv = _dinv(degp_ref)
    ssum = sp_ref[0] + sp_ref[1] + g2_ref[...]
    h2 = jnp.maximum(ssum * dinv + b2_ref[...], 0.0)
    b = batch_ref[0, 0, :]
    gids = lax.broadcasted_iota(jnp.int32, (NG, BLK), 0)
    oh = (b[None, :] == gids).astype(jnp.float32)

    @pl.when(i == 0)
    def _():
        sums[...] = jnp.zeros_like(sums)
        counts[...] = jnp.zeros_like(counts)

    sums[...] += jnp.dot(oh, h2, preferred_element_type=jnp.float32,
                         precision=lax.Precision.HIGHEST)
    counts[...] += jnp.broadcast_to(jnp.sum(oh, axis=1)[:, None], (NG, D))

    @pl.when(i == pl.num_programs(0) - 1)
    def _():
        pooled = sums[...] / jnp.maximum(counts[...], 1.0)
        out_ref[...] = jnp.dot(pooled, wl_ref[...],
                               preferred_element_type=jnp.float32,
                               precision=lax.Precision.HIGHEST) + bl_ref[...]


_tc1_call = pl.pallas_call(
    _tc1_body,
    grid=(NB,),
    in_specs=[
        pl.BlockSpec((BLK, D), lambda i: (i, 0)),
        pl.BlockSpec((D, D), lambda i: (0, 0)),
        pl.BlockSpec((BLK, NC), lambda i: (i, 0)),
    ],
    out_specs=pl.BlockSpec((BLK, D), lambda i: (i, 0)),
    out_shape=jax.ShapeDtypeStruct((N, D), jnp.float32),
)

_tc2_call = pl.pallas_call(
    _tc2_body,
    grid=(NB,),
    in_specs=[
        pl.BlockSpec((NC, BLK, D), lambda i: (0, i, 0)),
        pl.BlockSpec((BLK, D), lambda i: (i, 0)),
        pl.BlockSpec((BLK, NC), lambda i: (i, 0)),
        pl.BlockSpec((1, D), lambda i: (0, 0)),
        pl.BlockSpec((D, D), lambda i: (0, 0)),
    ],
    out_specs=pl.BlockSpec((BLK, D), lambda i: (i, 0)),
    out_shape=jax.ShapeDtypeStruct((N, D), jnp.float32),
)

_tc3_call = pl.pallas_call(
    _tc3_body,
    grid=(NB,),
    in_specs=[
        pl.BlockSpec((NC, BLK, D), lambda i: (0, i, 0)),
        pl.BlockSpec((BLK, D), lambda i: (i, 0)),
        pl.BlockSpec((BLK, NC), lambda i: (i, 0)),
        pl.BlockSpec((1, D), lambda i: (0, 0)),
        pl.BlockSpec((1, 1, BLK), lambda i: (i, 0, 0)),
        pl.BlockSpec((D, D), lambda i: (0, 0)),
        pl.BlockSpec((1, D), lambda i: (0, 0)),
    ],
    out_specs=pl.BlockSpec((NG, D), lambda i: (0, 0)),
    out_shape=jax.ShapeDtypeStruct((NG, D), jnp.float32),
    scratch_shapes=[
        pltpu.VMEM((NG, D), jnp.float32),
        pltpu.VMEM((NG, D), jnp.float32),
    ],
)


def kernel(x, edge_index, batch, W1, b1, W2, b2, Wl, bl):
    src = edge_index[0]
    dst = edge_index[1]
    degp = _deg_call(dst).reshape(NC, N).T
    g1 = _tc1_call(x, W1, degp)
    s1 = _scat_call(src, dst, g1).reshape(NC, NPAD, D)
    g2 = _tc2_call(s1, g1, degp, b1.reshape(1, D), W2)
    s2 = _scat_call(src, dst, g2).reshape(NC, NPAD, D)
    out = _tc3_call(s2, g2, degp, b2.reshape(1, D), batch.reshape(NB, 1, BLK),
                    Wl, bl.reshape(1, D))
    return out


# speedup vs baseline: 23.0012x; 1.0410x over previous
import functools

import jax
import jax.numpy as jnp
from jax import lax
from jax.experimental import pallas as pl
from jax.experimental.pallas import tpu as pltpu
from jax.experimental.pallas import tpu_sc as plsc

N = 10000
E = 320000
D = 128
NG = 64

NC = 2
NS = 16
NW = NC * NS
CHUNK = 128
NCH = E // CHUNK

NPAD = 10240
TROWS = NPAD // NS
ZR = 128

BLK = 1000
NB = N // BLK

DSL = (TROWS // 8) * 8
DEG_DSL = ((N // NS) // 8) * 8
DEG_DSL_LAST = N - DEG_DSL * (NS - 1)


def _wid():
    c = lax.axis_index("c")
    s = lax.axis_index("s")
    return s * NC + c, c, s


def _deg_body(dst_hbm, out_hbm, ones_v, didx, didx2, zbuf, acc1, sem_d,
              sem_d2):
    wid, c, s = _wid()
    for j in range(8):
        ones_v[pl.ds(j * 16, 16)] = jnp.ones((16,), jnp.float32)
    for j in range(DEG_DSL_LAST // 16):
        zbuf[pl.ds(j * 16, 16)] = jnp.zeros((16,), jnp.float32)

    @pl.when(s < NS - 1)
    def _():
        pltpu.sync_copy(zbuf.at[pl.ds(0, DEG_DSL)],
                        acc1.at[pl.ds(s * DEG_DSL, DEG_DSL)])

    @pl.when(s == NS - 1)
    def _():
        pltpu.sync_copy(zbuf, acc1.at[pl.ds((NS - 1) * DEG_DSL,
                                            DEG_DSL_LAST)])

    plsc.subcore_barrier()

    def body(i, carry):
        offa = (wid + (2 * i) * NW) * CHUNK
        offb = (wid + (2 * i + 1) * NW) * CHUNK
        cpa = pltpu.async_copy(dst_hbm.at[pl.ds(offa, CHUNK)], didx, sem_d)
        cpb = pltpu.async_copy(dst_hbm.at[pl.ds(offb, CHUNK)], didx2, sem_d2)
        cpa.wait()
        pltpu.sync_copy(ones_v, acc1.at[didx], add=True)
        cpb.wait()
        pltpu.sync_copy(ones_v, acc1.at[didx2], add=True)
        return carry

    lax.fori_loop(0, NCH // NW // 2, body, 0)

    @pl.when(wid < NCH % NW)
    def _():
        off = (wid + (NCH // NW) * NW) * CHUNK
        pltpu.sync_copy(dst_hbm.at[pl.ds(off, CHUNK)], didx)
        pltpu.sync_copy(ones_v, acc1.at[didx], add=True)

    plsc.subcore_barrier()

    @pl.when(s < NS - 1)
    def _():
        pltpu.sync_copy(acc1.at[pl.ds(s * DEG_DSL, DEG_DSL)],
                        zbuf.at[pl.ds(0, DEG_DSL)])
        pltpu.sync_copy(zbuf.at[pl.ds(0, DEG_DSL)],
                        out_hbm.at[pl.ds(c * N + s * DEG_DSL, DEG_DSL)])

    @pl.when(s == NS - 1)
    def _():
        pltpu.sync_copy(acc1.at[pl.ds((NS - 1) * DEG_DSL, DEG_DSL_LAST)],
                        zbuf)
        pltpu.sync_copy(zbuf,
                        out_hbm.at[pl.ds(c * N + (NS - 1) * DEG_DSL,
                                         DEG_DSL_LAST)])


_deg_call = pl.kernel(
    _deg_body,
    out_type=jax.ShapeDtypeStruct((NC * N,), jnp.float32),
    mesh=plsc.VectorSubcoreMesh(core_axis_name="c", subcore_axis_name="s"),
    scratch_types=[
        pltpu.VMEM((CHUNK,), jnp.float32),
        pltpu.VMEM((CHUNK,), jnp.int32),
        pltpu.VMEM((CHUNK,), jnp.int32),
        pltpu.VMEM((DEG_DSL_LAST,), jnp.float32),
        pltpu.VMEM_SHARED((N,), jnp.float32),
        pltpu.SemaphoreType.DMA,
        pltpu.SemaphoreType.DMA,
    ],
)


def _scat_body(src_hbm, dst_hbm, table_hbm, out_hbm,
               sidx, didx, sidx2, didx2, rows, rows2, acc,
               gsem, sem_s, sem_d, gsem2, sem_s2, sem_d2):
    wid, c, s = _wid()

    def zr(i, carry):
        for j in range(D // 16):
            rows[i, pl.ds(j * 16, 16)] = jnp.zeros((16,), jnp.float32)
        return carry

    lax.fori_loop(0, ZR, zr, 0)
    for j in range(TROWS // ZR):
        pltpu.sync_copy(rows, acc.at[pl.ds(s * TROWS + j * ZR, ZR)])
    plsc.subcore_barrier()

    def body(i, carry):
        offa = (wid + (2 * i) * NW) * CHUNK
        offb = (wid + (2 * i + 1) * NW) * CHUNK
        cpa_s = pltpu.async_copy(src_hbm.at[pl.ds(offa, CHUNK)], sidx, sem_s)
        cpa_d = pltpu.async_copy(dst_hbm.at[pl.ds(offa, CHUNK)], didx, sem_d)
        cpb_s = pltpu.async_copy(src_hbm.at[pl.ds(offb, CHUNK)], sidx2,
                                 sem_s2)
        cpb_d = pltpu.async_copy(dst_hbm.at[pl.ds(offb, CHUNK)], didx2,
                                 sem_d2)
        cpa_s.wait()
        ga = pltpu.async_copy(table_hbm.at[sidx], rows, gsem)
        cpb_s.wait()
        gb = pltpu.async_copy(table_hbm.at[sidx2], rows2, gsem2)
        ga.wait()
        cpa_d.wait()
        pltpu.sync_copy(rows, acc.at[didx], add=True)
        gb.wait()
        cpb_d.wait()
        pltpu.sync_copy(rows2, acc.at[didx2], add=True)
        return carry

    lax.fori_loop(0, NCH // NW // 2, body, 0)

    @pl.when(wid < NCH % NW)
    def _():
        off = (wid + (NCH // NW) * NW) * CHUNK
        pltpu.sync_copy(src_hbm.at[pl.ds(off, CHUNK)], sidx)
        pltpu.sync_copy(dst_hbm.at[pl.ds(off, CHUNK)], didx)
        pltpu.async_copy(table_hbm.at[sidx], rows, gsem).wait()
        pltpu.sync_copy(rows, acc.at[didx], add=True)

    plsc.subcore_barrier()
    for j in range(TROWS // ZR):
        buf = rows if j % 2 == 0 else rows2
        pltpu.sync_copy(acc.at[pl.ds(s * TROWS + j * ZR, ZR)], buf)
        pltpu.sync_copy(buf,
                        out_hbm.at[pl.ds(c * NPAD + s * TROWS + j * ZR, ZR)])


_scat_call = pl.kernel(
    _scat_body,
    out_type=jax.ShapeDtypeStruct((NC * NPAD, D), jnp.float32),
    mesh=plsc.VectorSubcoreMesh(core_axis_name="c", subcore_axis_name="s"),
    scratch_types=[
        pltpu.VMEM((CHUNK,), jnp.int32),
        pltpu.VMEM((CHUNK,), jnp.int32),
        pltpu.VMEM((CHUNK,), jnp.int32),
        pltpu.VMEM((CHUNK,), jnp.int32),
        pltpu.VMEM((CHUNK, D), jnp.float32),
        pltpu.VMEM((CHUNK, D), jnp.float32),
        pltpu.VMEM_SHARED((NPAD, D), jnp.float32),
        pltpu.SemaphoreType.DMA,
        pltpu.SemaphoreType.DMA,
        pltpu.SemaphoreType.DMA,
        pltpu.SemaphoreType.DMA,
        pltpu.SemaphoreType.DMA,
        pltpu.SemaphoreType.DMA,
    ],
)


def _dinv(degp_ref):
    return lax.rsqrt(jnp.sum(degp_ref[...], axis=1, keepdims=True) + 1.0)


def _tc1_body(x_ref, w_ref, degp_ref, g1_ref):
    dinv = _dinv(degp_ref)
    h = jnp.dot(x_ref[...], w_ref[...], preferred_element_type=jnp.float32,
                precision=lax.Precision.HIGHEST)
    g1_ref[...] = h * dinv


def _tc2_body(sp_ref, g1_ref, degp_ref, b1_ref, w2_ref, g2_ref):
    dinv = _dinv(degp_ref)
    ssum = sp_ref[0] + sp_ref[1] + g1_ref[...]
    out1 = jnp.maximum(ssum * dinv + b1_ref[...], 0.0)
    h2 = jnp.dot(out1, w2_ref[...], preferred_element_type=jnp.float32,
                 precision=lax.Precision.HIGHEST)
    g2_ref[...] = h2 * dinv


def _tc3_body(sp_ref, g2_ref, degp_ref, b2_ref, batch_ref, wl_ref, bl_ref,
              out_ref, sums, counts):
    i = pl.program_id(0)
    dinv = _dinv(degp_ref)
    ssum = sp_ref[0] + sp_ref[1] + g2_ref[...]
    h2 = jnp.maximum(ssum * dinv + b2_ref[...], 0.0)
    b = batch_ref[0, 0, :]
    gids = lax.broadcasted_iota(jnp.int32, (NG, BLK), 0)
    oh = (b[None, :] == gids).astype(jnp.float32)

    @pl.when(i == 0)
    def _():
        sums[...] = jnp.zeros_like(sums)
        counts[...] = jnp.zeros_like(counts)

    sums[...] += jnp.dot(oh, h2, preferred_element_type=jnp.float32,
                         precision=lax.Precision.HIGHEST)
    counts[...] += jnp.broadcast_to(jnp.sum(oh, axis=1)[:, None], (NG, D))

    @pl.when(i == pl.num_programs(0) - 1)
    def _():
        pooled = sums[...] / jnp.maximum(counts[...], 1.0)
        out_ref[...] = jnp.dot(pooled, wl_ref[...],
                               preferred_element_type=jnp.float32,
                               precision=lax.Precision.HIGHEST) + bl_ref[...]


_tc1_call = pl.pallas_call(
    _tc1_body,
    grid=(NB,),
    in_specs=[
        pl.BlockSpec((BLK, D), lambda i: (i, 0)),
        pl.BlockSpec((D, D), lambda i: (0, 0)),
        pl.BlockSpec((BLK, NC), lambda i: (i, 0)),
    ],
    out_specs=pl.BlockSpec((BLK, D), lambda i: (i, 0)),
    out_shape=jax.ShapeDtypeStruct((N, D), jnp.float32),
)

_tc2_call = pl.pallas_call(
    _tc2_body,
    grid=(NB,),
    in_specs=[
        pl.BlockSpec((NC, BLK, D), lambda i: (0, i, 0)),
        pl.BlockSpec((BLK, D), lambda i: (i, 0)),
        pl.BlockSpec((BLK, NC), lambda i: (i, 0)),
        pl.BlockSpec((1, D), lambda i: (0, 0)),
        pl.BlockSpec((D, D), lambda i: (0, 0)),
    ],
    out_specs=pl.BlockSpec((BLK, D), lambda i: (i, 0)),
    out_shape=jax.ShapeDtypeStruct((N, D), jnp.float32),
)

_tc3_call = pl.pallas_call(
    _tc3_body,
    grid=(NB,),
    in_specs=[
        pl.BlockSpec((NC, BLK, D), lambda i: (0, i, 0)),
        pl.BlockSpec((BLK, D), lambda i: (i, 0)),
        pl.BlockSpec((BLK, NC), lambda i: (i, 0)),
        pl.BlockSpec((1, D), lambda i: (0, 0)),
        pl.BlockSpec((1, 1, BLK), lambda i: (i, 0, 0)),
        pl.BlockSpec((D, D), lambda i: (0, 0)),
        pl.BlockSpec((1, D), lambda i: (0, 0)),
    ],
    out_specs=pl.BlockSpec((NG, D), lambda i: (0, 0)),
    out_shape=jax.ShapeDtypeStruct((NG, D), jnp.float32),
    scratch_shapes=[
        pltpu.VMEM((NG, D), jnp.float32),
        pltpu.VMEM((NG, D), jnp.float32),
    ],
)


def kernel(x, edge_index, batch, W1, b1, W2, b2, Wl, bl):
    src = edge_index[0]
    dst = edge_index[1]
    degp = _deg_call(dst).reshape(NC, N).T
    g1 = _tc1_call(x, W1, degp)
    s1 = _scat_call(src, dst, g1).reshape(NC, NPAD, D)
    g2 = _tc2_call(s1, g1, degp, b1.reshape(1, D), W2)
    s2 = _scat_call(src, dst, g2).reshape(NC, NPAD, D)
    out = _tc3_call(s2, g2, degp, b2.reshape(1, D), batch.reshape(NB, 1, BLK),
                    Wl, bl.reshape(1, D))
    return out


# speedup vs baseline: 24.2875x; 1.0559x over previous
import functools

import jax
import jax.numpy as jnp
from jax import lax
from jax.experimental import pallas as pl
from jax.experimental.pallas import tpu as pltpu
from jax.experimental.pallas import tpu_sc as plsc

N = 10000
E = 320000
D = 128
NG = 64

NC = 2
NS = 16
NW = NC * NS
CHUNK = 128
NCH = E // CHUNK

NPAD = 10112
TROWS = NPAD // NS
ZCH = [128, 128, 128, 128, 120]
ZR = 128

BLK = 1000
NB = N // BLK

DSL = (TROWS // 8) * 8
DEG_DSL = ((N // NS) // 8) * 8
DEG_DSL_LAST = N - DEG_DSL * (NS - 1)


def _wid():
    c = lax.axis_index("c")
    s = lax.axis_index("s")
    return s * NC + c, c, s


def _deg_body(dst_hbm, out_hbm, ones_v, didx, didx2, zbuf, acc1, sem_d,
              sem_d2):
    wid, c, s = _wid()
    for j in range(8):
        ones_v[pl.ds(j * 16, 16)] = jnp.ones((16,), jnp.float32)
    for j in range(DEG_DSL_LAST // 16):
        zbuf[pl.ds(j * 16, 16)] = jnp.zeros((16,), jnp.float32)

    @pl.when(s < NS - 1)
    def _():
        pltpu.sync_copy(zbuf.at[pl.ds(0, DEG_DSL)],
                        acc1.at[pl.ds(s * DEG_DSL, DEG_DSL)])

    @pl.when(s == NS - 1)
    def _():
        pltpu.sync_copy(zbuf, acc1.at[pl.ds((NS - 1) * DEG_DSL,
                                            DEG_DSL_LAST)])

    plsc.subcore_barrier()

    def body(i, carry):
        offa = (wid + (2 * i) * NW) * CHUNK
        offb = (wid + (2 * i + 1) * NW) * CHUNK
        cpa = pltpu.async_copy(dst_hbm.at[pl.ds(offa, CHUNK)], didx, sem_d)
        cpb = pltpu.async_copy(dst_hbm.at[pl.ds(offb, CHUNK)], didx2, sem_d2)
        cpa.wait()
        pltpu.sync_copy(ones_v, acc1.at[didx], add=True)
        cpb.wait()
        pltpu.sync_copy(ones_v, acc1.at[didx2], add=True)
        return carry

    lax.fori_loop(0, NCH // NW // 2, body, 0)

    @pl.when(wid < NCH % NW)
    def _():
        off = (wid + (NCH // NW) * NW) * CHUNK
        pltpu.sync_copy(dst_hbm.at[pl.ds(off, CHUNK)], didx)
        pltpu.sync_copy(ones_v, acc1.at[didx], add=True)

    plsc.subcore_barrier()

    @pl.when(s < NS - 1)
    def _():
        pltpu.sync_copy(acc1.at[pl.ds(s * DEG_DSL, DEG_DSL)],
                        zbuf.at[pl.ds(0, DEG_DSL)])
        pltpu.sync_copy(zbuf.at[pl.ds(0, DEG_DSL)],
                        out_hbm.at[pl.ds(c * N + s * DEG_DSL, DEG_DSL)])

    @pl.when(s == NS - 1)
    def _():
        pltpu.sync_copy(acc1.at[pl.ds((NS - 1) * DEG_DSL, DEG_DSL_LAST)],
                        zbuf)
        pltpu.sync_copy(zbuf,
                        out_hbm.at[pl.ds(c * N + (NS - 1) * DEG_DSL,
                                         DEG_DSL_LAST)])


_deg_call = pl.kernel(
    _deg_body,
    out_type=jax.ShapeDtypeStruct((NC * N,), jnp.float32),
    mesh=plsc.VectorSubcoreMesh(core_axis_name="c", subcore_axis_name="s"),
    scratch_types=[
        pltpu.VMEM((CHUNK,), jnp.float32),
        pltpu.VMEM((CHUNK,), jnp.int32),
        pltpu.VMEM((CHUNK,), jnp.int32),
        pltpu.VMEM((DEG_DSL_LAST,), jnp.float32),
        pltpu.VMEM_SHARED((N,), jnp.float32),
        pltpu.SemaphoreType.DMA,
        pltpu.SemaphoreType.DMA,
    ],
)


def _scat_body(src_hbm, dst_hbm, table_hbm, out_hbm,
               sidx, didx, sidx2, didx2, sidx3, didx3, rows, rows2, rows3,
               acc, gsem, sem_s, sem_d, gsem2, sem_s2, sem_d2, gsem3, sem_s3,
               sem_d3):
    wid, c, s = _wid()

    def zr(i, carry):
        for j in range(D // 16):
            rows[i, pl.ds(j * 16, 16)] = jnp.zeros((16,), jnp.float32)
        return carry

    lax.fori_loop(0, ZR, zr, 0)
    zo = 0
    for zc in ZCH:
        pltpu.sync_copy(rows.at[pl.ds(0, zc)],
                        acc.at[pl.ds(s * TROWS + zo, zc)])
        zo += zc
    plsc.subcore_barrier()

    def body(i, carry):
        offa = (wid + (3 * i) * NW) * CHUNK
        offb = (wid + (3 * i + 1) * NW) * CHUNK
        offc = (wid + (3 * i + 2) * NW) * CHUNK
        cpa_s = pltpu.async_copy(src_hbm.at[pl.ds(offa, CHUNK)], sidx, sem_s)
        cpa_d = pltpu.async_copy(dst_hbm.at[pl.ds(offa, CHUNK)], didx, sem_d)
        cpb_s = pltpu.async_copy(src_hbm.at[pl.ds(offb, CHUNK)], sidx2,
                                 sem_s2)
        cpb_d = pltpu.async_copy(dst_hbm.at[pl.ds(offb, CHUNK)], didx2,
                                 sem_d2)
        cpc_s = pltpu.async_copy(src_hbm.at[pl.ds(offc, CHUNK)], sidx3,
                                 sem_s3)
        cpc_d = pltpu.async_copy(dst_hbm.at[pl.ds(offc, CHUNK)], didx3,
                                 sem_d3)
        cpa_s.wait()
        ga = pltpu.async_copy(table_hbm.at[sidx], rows, gsem)
        cpb_s.wait()
        gb = pltpu.async_copy(table_hbm.at[sidx2], rows2, gsem2)
        cpc_s.wait()
        gc = pltpu.async_copy(table_hbm.at[sidx3], rows3, gsem3)
        ga.wait()
        cpa_d.wait()
        pltpu.sync_copy(rows, acc.at[didx], add=True)
        gb.wait()
        cpb_d.wait()
        pltpu.sync_copy(rows2, acc.at[didx2], add=True)
        gc.wait()
        cpc_d.wait()
        pltpu.sync_copy(rows3, acc.at[didx3], add=True)
        return carry

    lax.fori_loop(0, NCH // NW // 3, body, 0)

    @pl.when(wid < NCH % NW)
    def _():
        off = (wid + (NCH // NW) * NW) * CHUNK
        pltpu.sync_copy(src_hbm.at[pl.ds(off, CHUNK)], sidx)
        pltpu.sync_copy(dst_hbm.at[pl.ds(off, CHUNK)], didx)
        pltpu.async_copy(table_hbm.at[sidx], rows, gsem).wait()
        pltpu.sync_copy(rows, acc.at[didx], add=True)

    plsc.subcore_barrier()
    bufs = [rows, rows2, rows3]
    sems = [gsem, gsem2, gsem3]
    offs = [0, 128, 256, 384, 512]
    prev = pltpu.async_copy(acc.at[pl.ds(s * TROWS, ZCH[0])],
                            bufs[0].at[pl.ds(0, ZCH[0])], sems[0])
    for j in range(len(ZCH)):
        nxt = None
        if j + 1 < len(ZCH):
            nxt = pltpu.async_copy(
                acc.at[pl.ds(s * TROWS + offs[j + 1], ZCH[j + 1])],
                bufs[(j + 1) % 3].at[pl.ds(0, ZCH[j + 1])],
                sems[(j + 1) % 3])
        prev.wait()
        pltpu.sync_copy(bufs[j % 3].at[pl.ds(0, ZCH[j])],
                        out_hbm.at[pl.ds(c * NPAD + s * TROWS + offs[j],
                                         ZCH[j])])
        prev = nxt


_scat_call = pl.kernel(
    _scat_body,
    out_type=jax.ShapeDtypeStruct((NC * NPAD, D), jnp.float32),
    mesh=plsc.VectorSubcoreMesh(core_axis_name="c", subcore_axis_name="s"),
    scratch_types=[
        pltpu.VMEM((CHUNK,), jnp.int32),
        pltpu.VMEM((CHUNK,), jnp.int32),
        pltpu.VMEM((CHUNK,), jnp.int32),
        pltpu.VMEM((CHUNK,), jnp.int32),
        pltpu.VMEM((CHUNK,), jnp.int32),
        pltpu.VMEM((CHUNK,), jnp.int32),
        pltpu.VMEM((CHUNK, D), jnp.float32),
        pltpu.VMEM((CHUNK, D), jnp.float32),
        pltpu.VMEM((CHUNK, D), jnp.float32),
        pltpu.VMEM_SHARED((NPAD, D), jnp.float32),
        pltpu.SemaphoreType.DMA,
        pltpu.SemaphoreType.DMA,
        pltpu.SemaphoreType.DMA,
        pltpu.SemaphoreType.DMA,
        pltpu.SemaphoreType.DMA,
        pltpu.SemaphoreType.DMA,
        pltpu.SemaphoreType.DMA,
        pltpu.SemaphoreType.DMA,
        pltpu.SemaphoreType.DMA,
    ],
)


def _dinv(degp_ref):
    return lax.rsqrt(jnp.sum(degp_ref[...], axis=1, keepdims=True) + 1.0)


def _tc1_body(x_ref, w_ref, degp_ref, g1_ref):
    dinv = _dinv(degp_ref)
    h = jnp.dot(x_ref[...], w_ref[...], preferred_element_type=jnp.float32,
                precision=lax.Precision.HIGHEST)
    g1_ref[...] = h * dinv


def _tc2_body(sp_ref, g1_ref, degp_ref, b1_ref, w2_ref, g2_ref):
    dinv = _dinv(degp_ref)
    ssum = sp_ref[0] + sp_ref[1] + g1_ref[...]
    out1 = jnp.maximum(ssum * dinv + b1_ref[...], 0.0)
    h2 = jnp.dot(out1, w2_ref[...], preferred_element_type=jnp.float32,
                 precision=lax.Precision.HIGHEST)
    g2_ref[...] = h2 * dinv


def _tc3_body(sp_ref, g2_ref, degp_ref, b2_ref, batch_ref, wl_ref, bl_ref,
              out_ref, sums, counts):
    i = pl.program_id(0)
    dinv = _dinv(degp_ref)
    ssum = sp_ref[0] + sp_ref[1] + g2_ref[...]
    h2 = jnp.maximum(ssum * dinv + b2_ref[...], 0.0)
    b = batch_ref[0, 0, :]
    gids = lax.broadcasted_iota(jnp.int32, (NG, BLK), 0)
    oh = (b[None, :] == gids).astype(jnp.float32)

    @pl.when(i == 0)
    def _():
        sums[...] = jnp.zeros_like(sums)
        counts[...] = jnp.zeros_like(counts)

    sums[...] += jnp.dot(oh, h2, preferred_element_type=jnp.float32,
                         precision=lax.Precision.HIGHEST)
    counts[...] += jnp.broadcast_to(jnp.sum(oh, axis=1)[:, None], (NG, D))

    @pl.when(i == pl.num_programs(0) - 1)
    def _():
        pooled = sums[...] / jnp.maximum(counts[...], 1.0)
        out_ref[...] = jnp.dot(pooled, wl_ref[...],
                               preferred_element_type=jnp.float32,
                               precision=lax.Precision.HIGHEST) + bl_ref[...]


_tc1_call = pl.pallas_call(
    _tc1_body,
    grid=(NB,),
    in_specs=[
        pl.BlockSpec((BLK, D), lambda i: (i, 0)),
        pl.BlockSpec((D, D), lambda i: (0, 0)),
        pl.BlockSpec((BLK, NC), lambda i: (i, 0)),
    ],
    out_specs=pl.BlockSpec((BLK, D), lambda i: (i, 0)),
    out_shape=jax.ShapeDtypeStruct((N, D), jnp.float32),
)

_tc2_call = pl.pallas_call(
    _tc2_body,
    grid=(NB,),
    in_specs=[
        pl.BlockSpec((NC, BLK, D), lambda i: (0, i, 0)),
        pl.BlockSpec((BLK, D), lambda i: (i, 0)),
        pl.BlockSpec((BLK, NC), lambda i: (i, 0)),
        pl.BlockSpec((1, D), lambda i: (0, 0)),
        pl.BlockSpec((D, D), lambda i: (0, 0)),
    ],
    out_specs=pl.BlockSpec((BLK, D), lambda i: (i, 0)),
    out_shape=jax.ShapeDtypeStruct((N, D), jnp.float32),
)

_tc3_call = pl.pallas_call(
    _tc3_body,
    grid=(NB,),
    in_specs=[
        pl.BlockSpec((NC, BLK, D), lambda i: (0, i, 0)),
        pl.BlockSpec((BLK, D), lambda i: (i, 0)),
        pl.BlockSpec((BLK, NC), lambda i: (i, 0)),
        pl.BlockSpec((1, D), lambda i: (0, 0)),
        pl.BlockSpec((1, 1, BLK), lambda i: (i, 0, 0)),
        pl.BlockSpec((D, D), lambda i: (0, 0)),
        pl.BlockSpec((1, D), lambda i: (0, 0)),
    ],
    out_specs=pl.BlockSpec((NG, D), lambda i: (0, 0)),
    out_shape=jax.ShapeDtypeStruct((NG, D), jnp.float32),
    scratch_shapes=[
        pltpu.VMEM((NG, D), jnp.float32),
        pltpu.VMEM((NG, D), jnp.float32),
    ],
)


def kernel(x, edge_index, batch, W1, b1, W2, b2, Wl, bl):
    src = edge_index[0]
    dst = edge_index[1]
    degp = _deg_call(dst).reshape(NC, N).T
    g1 = _tc1_call(x, W1, degp)
    s1 = _scat_call(src, dst, g1).reshape(NC, NPAD, D)
    g2 = _tc2_call(s1, g1, degp, b1.reshape(1, D), W2)
    s2 = _scat_call(src, dst, g2).reshape(NC, NPAD, D)
    out = _tc3_call(s2, g2, degp, b2.reshape(1, D), batch.reshape(NB, 1, BLK),
                    Wl, bl.reshape(1, D))
    return out


# speedup vs baseline: 24.4785x; 1.0079x over previous
import functools

import jax
import jax.numpy as jnp
from jax import lax
from jax.experimental import pallas as pl
from jax.experimental.pallas import tpu as pltpu
from jax.experimental.pallas import tpu_sc as plsc

N = 10000
E = 320000
D = 128
NG = 64

NC = 2
NS = 16
NW = NC * NS
CHUNK = 128
NCH = E // CHUNK

NPAD = 10112
TROWS = NPAD // NS
ZCH = [128, 128, 128, 128, 120]
ZR = 128

BLK = 1000
NB = N // BLK

DSL = (TROWS // 8) * 8
DEG_DSL = ((N // NS) // 8) * 8
DEG_DSL_LAST = N - DEG_DSL * (NS - 1)


def _wid():
    c = lax.axis_index("c")
    s = lax.axis_index("s")
    return s * NC + c, c, s


def _deg_body(dst_hbm, out_hbm, ones_v, didx, didx2, zbuf, acc1, sem_d,
              sem_d2):
    wid, c, s = _wid()
    for j in range(8):
        ones_v[pl.ds(j * 16, 16)] = jnp.ones((16,), jnp.float32)
    for j in range(DEG_DSL_LAST // 16):
        zbuf[pl.ds(j * 16, 16)] = jnp.zeros((16,), jnp.float32)

    @pl.when(s < NS - 1)
    def _():
        pltpu.sync_copy(zbuf.at[pl.ds(0, DEG_DSL)],
                        acc1.at[pl.ds(s * DEG_DSL, DEG_DSL)])

    @pl.when(s == NS - 1)
    def _():
        pltpu.sync_copy(zbuf, acc1.at[pl.ds((NS - 1) * DEG_DSL,
                                            DEG_DSL_LAST)])

    plsc.subcore_barrier()

    def body(i, carry):
        offa = (wid + (2 * i) * NW) * CHUNK
        offb = (wid + (2 * i + 1) * NW) * CHUNK
        cpa = pltpu.async_copy(dst_hbm.at[pl.ds(offa, CHUNK)], didx, sem_d)
        cpb = pltpu.async_copy(dst_hbm.at[pl.ds(offb, CHUNK)], didx2, sem_d2)
        cpa.wait()
        pltpu.sync_copy(ones_v, acc1.at[didx], add=True)
        cpb.wait()
        pltpu.sync_copy(ones_v, acc1.at[didx2], add=True)
        return carry

    lax.fori_loop(0, NCH // NW // 2, body, 0)

    @pl.when(wid < NCH % NW)
    def _():
        off = (wid + (NCH // NW) * NW) * CHUNK
        pltpu.sync_copy(dst_hbm.at[pl.ds(off, CHUNK)], didx)
        pltpu.sync_copy(ones_v, acc1.at[didx], add=True)

    plsc.subcore_barrier()

    @pl.when(s < NS - 1)
    def _():
        pltpu.sync_copy(acc1.at[pl.ds(s * DEG_DSL, DEG_DSL)],
                        zbuf.at[pl.ds(0, DEG_DSL)])
        pltpu.sync_copy(zbuf.at[pl.ds(0, DEG_DSL)],
                        out_hbm.at[pl.ds(c * N + s * DEG_DSL, DEG_DSL)])

    @pl.when(s == NS - 1)
    def _():
        pltpu.sync_copy(acc1.at[pl.ds((NS - 1) * DEG_DSL, DEG_DSL_LAST)],
                        zbuf)
        pltpu.sync_copy(zbuf,
                        out_hbm.at[pl.ds(c * N + (NS - 1) * DEG_DSL,
                                         DEG_DSL_LAST)])


_deg_call = pl.kernel(
    _deg_body,
    out_type=jax.ShapeDtypeStruct((NC * N,), jnp.float32),
    mesh=plsc.VectorSubcoreMesh(core_axis_name="c", subcore_axis_name="s"),
    scratch_types=[
        pltpu.VMEM((CHUNK,), jnp.float32),
        pltpu.VMEM((CHUNK,), jnp.int32),
        pltpu.VMEM((CHUNK,), jnp.int32),
        pltpu.VMEM((DEG_DSL_LAST,), jnp.float32),
        pltpu.VMEM_SHARED((N,), jnp.float32),
        pltpu.SemaphoreType.DMA,
        pltpu.SemaphoreType.DMA,
    ],
)


def _scat_body(src_hbm, dst_hbm, table_hbm, out_hbm,
               sidx, didx, sidx2, didx2, sidx3, didx3, rows, rows2, rows3,
               acc, gsem, sem_s, sem_d, gsem2, sem_s2, sem_d2, gsem3, sem_s3,
               sem_d3):
    wid, c, s = _wid()

    def zr(i, carry):
        for j in range(D // 16):
            rows[i, pl.ds(j * 16, 16)] = jnp.zeros((16,), jnp.float32)
        return carry

    lax.fori_loop(0, ZR, zr, 0)
    zo = 0
    for zc in ZCH:
        pltpu.sync_copy(rows.at[pl.ds(0, zc)],
                        acc.at[pl.ds(s * TROWS + zo, zc)])
        zo += zc
    plsc.subcore_barrier()

    def body(i, carry):
        offa = (wid + (3 * i) * NW) * CHUNK
        offb = (wid + (3 * i + 1) * NW) * CHUNK
        offc = (wid + (3 * i + 2) * NW) * CHUNK
        cpa_s = pltpu.async_copy(src_hbm.at[pl.ds(offa, CHUNK)], sidx, sem_s)
        cpa_d = pltpu.async_copy(dst_hbm.at[pl.ds(offa, CHUNK)], didx, sem_d)
        cpb_s = pltpu.async_copy(src_hbm.at[pl.ds(offb, CHUNK)], sidx2,
                                 sem_s2)
        cpb_d = pltpu.async_copy(dst_hbm.at[pl.ds(offb, CHUNK)], didx2,
                                 sem_d2)
        cpc_s = pltpu.async_copy(src_hbm.at[pl.ds(offc, CHUNK)], sidx3,
                                 sem_s3)
        cpc_d = pltpu.async_copy(dst_hbm.at[pl.ds(offc, CHUNK)], didx3,
                                 sem_d3)
        cpa_s.wait()
        ga = pltpu.async_copy(table_hbm.at[sidx], rows, gsem)
        cpb_s.wait()
        gb = pltpu.async_copy(table_hbm.at[sidx2], rows2, gsem2)
        cpc_s.wait()
        gc = pltpu.async_copy(table_hbm.at[sidx3], rows3, gsem3)
        ga.wait()
        cpa_d.wait()
        pltpu.sync_copy(rows, acc.at[didx], add=True)
        gb.wait()
        cpb_d.wait()
        pltpu.sync_copy(rows2, acc.at[didx2], add=True)
        gc.wait()
        cpc_d.wait()
        pltpu.sync_copy(rows3, acc.at[didx3], add=True)
        return carry

    lax.fori_loop(0, NCH // NW // 3, body, 0)

    @pl.when(wid < NCH % NW)
    def _():
        off = (wid + (NCH // NW) * NW) * CHUNK
        pltpu.sync_copy(src_hbm.at[pl.ds(off, CHUNK)], sidx)
        pltpu.sync_copy(dst_hbm.at[pl.ds(off, CHUNK)], didx)
        pltpu.async_copy(table_hbm.at[sidx], rows, gsem).wait()
        pltpu.sync_copy(rows, acc.at[didx], add=True)

    plsc.subcore_barrier()
    bufs = [rows, rows2, rows3]
    sems = [gsem, gsem2, gsem3]
    offs = [0, 128, 256, 384, 512]
    prev = pltpu.async_copy(acc.at[pl.ds(s * TROWS, ZCH[0])],
                            bufs[0].at[pl.ds(0, ZCH[0])], sems[0])
    for j in range(len(ZCH)):
        nxt = None
        if j + 1 < len(ZCH):
            nxt = pltpu.async_copy(
                acc.at[pl.ds(s * TROWS + offs[j + 1], ZCH[j + 1])],
                bufs[(j + 1) % 3].at[pl.ds(0, ZCH[j + 1])],
                sems[(j + 1) % 3])
        prev.wait()
        pltpu.sync_copy(bufs[j % 3].at[pl.ds(0, ZCH[j])],
                        out_hbm.at[pl.ds(c * NPAD + s * TROWS + offs[j],
                                         ZCH[j])])
        prev = nxt


_scat_call = pl.kernel(
    _scat_body,
    out_type=jax.ShapeDtypeStruct((NC * NPAD, D), jnp.float32),
    mesh=plsc.VectorSubcoreMesh(core_axis_name="c", subcore_axis_name="s"),
    scratch_types=[
        pltpu.VMEM((CHUNK,), jnp.int32),
        pltpu.VMEM((CHUNK,), jnp.int32),
        pltpu.VMEM((CHUNK,), jnp.int32),
        pltpu.VMEM((CHUNK,), jnp.int32),
        pltpu.VMEM((CHUNK,), jnp.int32),
        pltpu.VMEM((CHUNK,), jnp.int32),
        pltpu.VMEM((CHUNK, D), jnp.float32),
        pltpu.VMEM((CHUNK, D), jnp.float32),
        pltpu.VMEM((CHUNK, D), jnp.float32),
        pltpu.VMEM_SHARED((NPAD, D), jnp.float32),
        pltpu.SemaphoreType.DMA,
        pltpu.SemaphoreType.DMA,
        pltpu.SemaphoreType.DMA,
        pltpu.SemaphoreType.DMA,
        pltpu.SemaphoreType.DMA,
        pltpu.SemaphoreType.DMA,
        pltpu.SemaphoreType.DMA,
        pltpu.SemaphoreType.DMA,
        pltpu.SemaphoreType.DMA,
    ],
)


def _dinv(degp_ref):
    return lax.rsqrt(jnp.sum(degp_ref[...], axis=1, keepdims=True) + 1.0)


def _mm_body(x_ref, w_ref, h_ref):
    h_ref[...] = jnp.dot(x_ref[...], w_ref[...],
                         preferred_element_type=jnp.float32,
                         precision=lax.Precision.HIGHEST)


def _scale_body(h_ref, degp_ref, g1_ref):
    g1_ref[...] = h_ref[...] * _dinv(degp_ref)


def _tc2_body(sp_ref, g1_ref, degp_ref, b1_ref, w2_ref, g2_ref):
    dinv = _dinv(degp_ref)
    ssum = sp_ref[0] + sp_ref[1] + g1_ref[...]
    out1 = jnp.maximum(ssum * dinv + b1_ref[...], 0.0)
    h2 = jnp.dot(out1, w2_ref[...], preferred_element_type=jnp.float32,
                 precision=lax.Precision.HIGHEST)
    g2_ref[...] = h2 * dinv


def _tc3_body(sp_ref, g2_ref, degp_ref, b2_ref, batch_ref, wl_ref, bl_ref,
              out_ref, sums, counts):
    i = pl.program_id(0)
    dinv = _dinv(degp_ref)
    ssum = sp_ref[0] + sp_ref[1] + g2_ref[...]
    h2 = jnp.maximum(ssum * dinv + b2_ref[...], 0.0)
    b = batch_ref[0, 0, :]
    gids = lax.broadcasted_iota(jnp.int32, (NG, BLK), 0)
    oh = (b[None, :] == gids).astype(jnp.float32)

    @pl.when(i == 0)
    def _():
        sums[...] = jnp.zeros_like(sums)
        counts[...] = jnp.zeros_like(counts)

    sums[...] += jnp.dot(oh, h2, preferred_element_type=jnp.float32,
                         precision=lax.Precision.HIGHEST)
    counts[...] += jnp.broadcast_to(jnp.sum(oh, axis=1)[:, None], (NG, D))

    @pl.when(i == pl.num_programs(0) - 1)
    def _():
        pooled = sums[...] / jnp.maximum(counts[...], 1.0)
        out_ref[...] = jnp.dot(pooled, wl_ref[...],
                               preferred_element_type=jnp.float32,
                               precision=lax.Precision.HIGHEST) + bl_ref[...]


_mm_call = pl.pallas_call(
    _mm_body,
    grid=(NB,),
    in_specs=[
        pl.BlockSpec((BLK, D), lambda i: (i, 0)),
        pl.BlockSpec((D, D), lambda i: (0, 0)),
    ],
    out_specs=pl.BlockSpec((BLK, D), lambda i: (i, 0)),
    out_shape=jax.ShapeDtypeStruct((N, D), jnp.float32),
)

_scale_call = pl.pallas_call(
    _scale_body,
    grid=(NB,),
    in_specs=[
        pl.BlockSpec((BLK, D), lambda i: (i, 0)),
        pl.BlockSpec((BLK, NC), lambda i: (i, 0)),
    ],
    out_specs=pl.BlockSpec((BLK, D), lambda i: (i, 0)),
    out_shape=jax.ShapeDtypeStruct((N, D), jnp.float32),
)

_tc2_call = pl.pallas_call(
    _tc2_body,
    grid=(NB,),
    in_specs=[
        pl.BlockSpec((NC, BLK, D), lambda i: (0, i, 0)),
        pl.BlockSpec((BLK, D), lambda i: (i, 0)),
        pl.BlockSpec((BLK, NC), lambda i: (i, 0)),
        pl.BlockSpec((1, D), lambda i: (0, 0)),
        pl.BlockSpec((D, D), lambda i: (0, 0)),
    ],
    out_specs=pl.BlockSpec((BLK, D), lambda i: (i, 0)),
    out_shape=jax.ShapeDtypeStruct((N, D), jnp.float32),
)

_tc3_call = pl.pallas_call(
    _tc3_body,
    grid=(NB,),
    in_specs=[
        pl.BlockSpec((NC, BLK, D), lambda i: (0, i, 0)),
        pl.BlockSpec((BLK, D), lambda i: (i, 0)),
        pl.BlockSpec((BLK, NC), lambda i: (i, 0)),
        pl.BlockSpec((1, D), lambda i: (0, 0)),
        pl.BlockSpec((1, 1, BLK), lambda i: (i, 0, 0)),
        pl.BlockSpec((D, D), lambda i: (0, 0)),
        pl.BlockSpec((1, D), lambda i: (0, 0)),
    ],
    out_specs=pl.BlockSpec((NG, D), lambda i: (0, 0)),
    out_shape=jax.ShapeDtypeStruct((NG, D), jnp.float32),
    scratch_shapes=[
        pltpu.VMEM((NG, D), jnp.float32),
        pltpu.VMEM((NG, D), jnp.float32),
    ],
)


def kernel(x, edge_index, batch, W1, b1, W2, b2, Wl, bl):
    src = edge_index[0]
    dst = edge_index[1]
    h1 = _mm_call(x, W1)
    degp = _deg_call(dst).reshape(NC, N).T
    g1 = _scale_call(h1, degp)
    s1 = _scat_call(src, dst, g1).reshape(NC, NPAD, D)
    g2 = _tc2_call(s1, g1, degp, b1.reshape(1, D), W2)
    s2 = _scat_call(src, dst, g2).reshape(NC, NPAD, D)
    out = _tc3_call(s2, g2, degp, b2.reshape(1, D), batch.reshape(NB, 1, BLK),
                    Wl, bl.reshape(1, D))
    return out
